# Initial kernel scaffold; baseline (speedup 1.0000x reference)
import jax
import jax.numpy as jnp
from jax.experimental import pallas as pl


def kernel(distance):
    raise NotImplementedError("write your pallas kernel here")


# speedup vs baseline: 6.7156x; 6.7156x over previous
import functools

import jax
import jax.numpy as jnp
from jax import lax
from jax.experimental import pallas as pl
from jax.experimental.pallas import tpu as pltpu
from jax.experimental.pallas import tpu_sc as plsc

_NUM_CORES = 2
_NUM_SUBCORES = 16
_LANES = 16
_NW = _NUM_CORES * _NUM_SUBCORES


def _index_body(d_ref, o_ref):
    d = d_ref[0]
    m = d.shape[-1]
    cs = jnp.sum(d, axis=0, keepdims=True)
    w = lax.broadcasted_iota(jnp.int32, (1, m), 1).astype(jnp.float32)
    s1 = jnp.sum(cs * w)
    s0 = jnp.sum(cs)
    scale = (s1 / s0) / jnp.float32(0.5 * m)
    idx = (scale * w).astype(jnp.int32)
    o_ref[0] = jnp.clip(idx, 0, m - 1)


def _compute_indices(distance):
    b, l, m = distance.shape
    out = pl.pallas_call(
        _index_body,
        grid=(b,),
        in_specs=[pl.BlockSpec((1, l, m), lambda i: (i, 0, 0))],
        out_specs=pl.BlockSpec((1, 1, m), lambda i: (i, 0, 0)),
        out_shape=jax.ShapeDtypeStruct((b, 1, m), jnp.int32),
    )(distance)
    return out.reshape(b * m)


def _make_sc_gather(B, L, M, R):
    rows_per_w = L // _NW
    cpb = rows_per_w // R
    nch = B * cpb
    csz = R * M

    mesh = plsc.VectorSubcoreMesh(
        core_axis_name="c", subcore_axis_name="s",
        num_cores=_NUM_CORES, num_subcores=_NUM_SUBCORES)

    @functools.partial(
        pl.kernel,
        mesh=mesh,
        compiler_params=pltpu.CompilerParams(needs_layout_passes=False),
        out_type=jax.ShapeDtypeStruct((B * L, M), jnp.float32),
        scratch_types=[
            pltpu.VMEM((B * M,), jnp.int32),
            pltpu.VMEM((R, M), jnp.float32),
            pltpu.VMEM((R, M), jnp.float32),
            pltpu.VMEM((R, M), jnp.float32),
            pltpu.VMEM((R, M), jnp.float32),
            pltpu.SemaphoreType.DMA,
            pltpu.SemaphoreType.DMA,
            pltpu.SemaphoreType.DMA,
            pltpu.SemaphoreType.DMA,
        ],
    )
    def sc_gather(d_hbm, idx_hbm, out_hbm, idx_v, in0, in1, out0, out1,
                  si0, si1, so0, so1):
        wid = lax.axis_index("s") * _NUM_CORES + lax.axis_index("c")
        row_base = wid * rows_per_w

        def chunk_off(t):
            bt = t // cpb
            ct = t % cpb
            return bt * L + row_base + ct * R

        def start_in(t, buf, sem):
            pltpu.make_async_copy(
                d_hbm.at[pl.ds(chunk_off(t), R)], buf, sem).start()

        def wait_in(t, buf, sem):
            pltpu.make_async_copy(
                d_hbm.at[pl.ds(chunk_off(t), R)], buf, sem).wait()

        def start_out(t, buf, sem):
            pltpu.make_async_copy(
                buf, out_hbm.at[pl.ds(chunk_off(t), R)], sem).start()

        def wait_out(t, buf, sem):
            pltpu.make_async_copy(
                buf, out_hbm.at[pl.ds(chunk_off(t), R)], sem).wait()

        pltpu.sync_copy(idx_hbm, idx_v)

        start_in(0, in0, si0)

        def do_chunk(t, inb, outb):
            bt = t // cpb
            ibase = bt * M

            def g_body(g, carry):
                col = idx_v[pl.ds(ibase + g * _LANES, _LANES)]
                for r in range(R):
                    row = jnp.full((_LANES,), r, jnp.int32)
                    v = plsc.load_gather(inb, [row, col])
                    outb[r, pl.ds(g * _LANES, _LANES)] = v
                return carry

            lax.fori_loop(0, M // _LANES, g_body, 0)

        def loop_body(t, carry):
            def phase(inb, outb, sin, sout, nxt_in, nxt_sin):
                @pl.when(t >= 2)
                def _():
                    wait_out(t - 2, outb, sout)

                @pl.when(t + 1 < nch)
                def _():
                    start_in(t + 1, nxt_in, nxt_sin)

                wait_in(t, inb, sin)
                do_chunk(t, inb, outb)
                start_out(t, outb, sout)

            @pl.when(t % 2 == 0)
            def _():
                phase(in0, out0, si0, so0, in1, si1)

            @pl.when(t % 2 == 1)
            def _():
                phase(in1, out1, si1, so1, in0, si0)

            return carry

        lax.fori_loop(0, nch, loop_body, 0)
        wait_out(nch - 2, out0, so0)
        wait_out(nch - 1, out1, so1)

    return sc_gather


def kernel(distance):
    original_shape = distance.shape
    d3 = distance.reshape(-1, original_shape[-2], original_shape[-1])
    B, L, M = d3.shape
    idx_flat = _compute_indices(d3)
    gather = _make_sc_gather(B, L, M, R=32)
    out_flat = gather(d3.reshape(B * L, M), idx_flat)
    return out_flat.reshape(original_shape)


# speedup vs baseline: 12.8273x; 1.9101x over previous
import functools

import jax
import jax.numpy as jnp
from jax import lax
from jax.experimental import pallas as pl
from jax.experimental.pallas import tpu as pltpu
from jax.experimental.pallas import tpu_sc as plsc

_NUM_CORES = 2
_NUM_SUBCORES = 16
_LANES = 16
_NW = _NUM_CORES * _NUM_SUBCORES


def _index_body(d_ref, o_ref):
    d = d_ref[0]
    m = d.shape[-1]
    cs = jnp.sum(d, axis=0, keepdims=True)
    w = lax.broadcasted_iota(jnp.int32, (1, m), 1).astype(jnp.float32)
    s1 = jnp.sum(cs * w)
    s0 = jnp.sum(cs)
    scale = (s1 / s0) / jnp.float32(0.5 * m)
    idx = (scale * w).astype(jnp.int32)
    o_ref[0] = jnp.clip(idx, 0, m - 1)


def _compute_indices(distance):
    b, l, m = distance.shape
    out = pl.pallas_call(
        _index_body,
        grid=(b,),
        in_specs=[pl.BlockSpec((1, l, m), lambda i: (i, 0, 0))],
        out_specs=pl.BlockSpec((1, 1, m), lambda i: (i, 0, 0)),
        out_shape=jax.ShapeDtypeStruct((b, 1, m), jnp.int32),
    )(distance)
    return out.reshape(b * m)


def _make_sc_gather(B, L, M, R):
    rows_per_w = L // _NW
    cpb = rows_per_w // R
    nch = B * cpb
    csz = R * M

    mesh = plsc.VectorSubcoreMesh(
        core_axis_name="c", subcore_axis_name="s",
        num_cores=_NUM_CORES, num_subcores=_NUM_SUBCORES)

    @functools.partial(
        pl.kernel,
        mesh=mesh,
        compiler_params=pltpu.CompilerParams(needs_layout_passes=False),
        out_type=jax.ShapeDtypeStruct((B * L, M), jnp.float32),
        scratch_types=[
            pltpu.VMEM((B * M,), jnp.int32),
            pltpu.VMEM((R, M), jnp.float32),
            pltpu.VMEM((R, M), jnp.float32),
            pltpu.VMEM((R, M), jnp.float32),
            pltpu.VMEM((R, M), jnp.float32),
            pltpu.SemaphoreType.DMA,
            pltpu.SemaphoreType.DMA,
            pltpu.SemaphoreType.DMA,
            pltpu.SemaphoreType.DMA,
        ],
    )
    def sc_gather(d_hbm, idx_hbm, out_hbm, idx_v, in0, in1, out0, out1,
                  si0, si1, so0, so1):
        wid = lax.axis_index("s") * _NUM_CORES + lax.axis_index("c")
        row_base = wid * rows_per_w

        def chunk_off(t):
            bt = t // cpb
            ct = t % cpb
            return bt * L + row_base + ct * R

        def start_in(t, buf, sem):
            pltpu.make_async_copy(
                d_hbm.at[pl.ds(chunk_off(t), R)], buf, sem).start()

        def wait_in(t, buf, sem):
            pltpu.make_async_copy(
                d_hbm.at[pl.ds(chunk_off(t), R)], buf, sem).wait()

        def start_out(t, buf, sem):
            pltpu.make_async_copy(
                buf, out_hbm.at[pl.ds(chunk_off(t), R)], sem).start()

        def wait_out(t, buf, sem):
            pltpu.make_async_copy(
                buf, out_hbm.at[pl.ds(chunk_off(t), R)], sem).wait()

        pltpu.sync_copy(idx_hbm, idx_v)

        start_in(0, in0, si0)

        def do_chunk(t, inb, outb):
            bt = t // cpb
            ibase = bt * M

            def g_body(g, carry):
                col = idx_v[pl.ds(ibase + g * _LANES, _LANES)]
                grp = 8
                for r0 in range(0, R, grp):
                    vs = []
                    for r in range(r0, r0 + grp):
                        row = jnp.full((_LANES,), r, jnp.int32)
                        vs.append(plsc.load_gather(inb, [row, col]))
                    for i, r in enumerate(range(r0, r0 + grp)):
                        outb[r, pl.ds(g * _LANES, _LANES)] = vs[i]
                return carry

            lax.fori_loop(0, M // _LANES, g_body, 0)

        def loop_body(t, carry):
            def phase(inb, outb, sin, sout, nxt_in, nxt_sin):
                @pl.when(t >= 2)
                def _():
                    wait_out(t - 2, outb, sout)

                @pl.when(t + 1 < nch)
                def _():
                    start_in(t + 1, nxt_in, nxt_sin)

                wait_in(t, inb, sin)
                do_chunk(t, inb, outb)
                start_out(t, outb, sout)

            @pl.when(t % 2 == 0)
            def _():
                phase(in0, out0, si0, so0, in1, si1)

            @pl.when(t % 2 == 1)
            def _():
                phase(in1, out1, si1, so1, in0, si0)

            return carry

        lax.fori_loop(0, nch, loop_body, 0)
        wait_out(nch - 2, out0, so0)
        wait_out(nch - 1, out1, so1)

    return sc_gather


def kernel(distance):
    original_shape = distance.shape
    d3 = distance.reshape(-1, original_shape[-2], original_shape[-1])
    B, L, M = d3.shape
    idx_flat = _compute_indices(d3)
    gather = _make_sc_gather(B, L, M, R=32)
    out_flat = gather(d3.reshape(B * L, M), idx_flat)
    return out_flat.reshape(original_shape)


# speedup vs baseline: 12.8607x; 1.0026x over previous
import functools

import jax
import jax.numpy as jnp
from jax import lax
from jax.experimental import pallas as pl
from jax.experimental.pallas import tpu as pltpu
from jax.experimental.pallas import tpu_sc as plsc

_NUM_CORES = 2
_NUM_SUBCORES = 16
_LANES = 16
_NW = _NUM_CORES * _NUM_SUBCORES


def _index_body(d_ref, o_ref):
    d = d_ref[0]
    m = d.shape[-1]
    cs = jnp.sum(d, axis=0, keepdims=True)
    w = lax.broadcasted_iota(jnp.int32, (1, m), 1).astype(jnp.float32)
    s1 = jnp.sum(cs * w)
    s0 = jnp.sum(cs)
    scale = (s1 / s0) / jnp.float32(0.5 * m)
    idx = (scale * w).astype(jnp.int32)
    o_ref[0] = jnp.clip(idx, 0, m - 1)


def _compute_indices(distance):
    b, l, m = distance.shape
    out = pl.pallas_call(
        _index_body,
        grid=(b,),
        in_specs=[pl.BlockSpec((1, l, m), lambda i: (i, 0, 0))],
        out_specs=pl.BlockSpec((1, 1, m), lambda i: (i, 0, 0)),
        out_shape=jax.ShapeDtypeStruct((b, 1, m), jnp.int32),
    )(distance)
    return out.reshape(b * m)


def _make_sc_gather(B, L, M, R):
    rows_per_w = L // _NW
    cpb = rows_per_w // R
    nch = B * cpb
    csz = R * M

    mesh = plsc.VectorSubcoreMesh(
        core_axis_name="c", subcore_axis_name="s",
        num_cores=_NUM_CORES, num_subcores=_NUM_SUBCORES)

    @functools.partial(
        pl.kernel,
        mesh=mesh,
        compiler_params=pltpu.CompilerParams(needs_layout_passes=False),
        out_type=jax.ShapeDtypeStruct((B * L, M), jnp.float32),
        scratch_types=[
            pltpu.VMEM((B * M,), jnp.int32),
            pltpu.VMEM((R, M), jnp.float32),
            pltpu.VMEM((R, M), jnp.float32),
            pltpu.VMEM((R, M), jnp.float32),
            pltpu.VMEM((R, M), jnp.float32),
            pltpu.SemaphoreType.DMA,
            pltpu.SemaphoreType.DMA,
            pltpu.SemaphoreType.DMA,
            pltpu.SemaphoreType.DMA,
        ],
    )
    def sc_gather(d_hbm, idx_hbm, out_hbm, idx_v, in0, in1, out0, out1,
                  si0, si1, so0, so1):
        wid = lax.axis_index("s") * _NUM_CORES + lax.axis_index("c")
        row_base = wid * rows_per_w

        def chunk_off(t):
            bt = t // cpb
            ct = t % cpb
            return bt * L + row_base + ct * R

        def start_in(t, buf, sem):
            pltpu.make_async_copy(
                d_hbm.at[pl.ds(chunk_off(t), R)], buf, sem).start()

        def wait_in(t, buf, sem):
            pltpu.make_async_copy(
                d_hbm.at[pl.ds(chunk_off(t), R)], buf, sem).wait()

        def start_out(t, buf, sem):
            pltpu.make_async_copy(
                buf, out_hbm.at[pl.ds(chunk_off(t), R)], sem).start()

        def wait_out(t, buf, sem):
            pltpu.make_async_copy(
                buf, out_hbm.at[pl.ds(chunk_off(t), R)], sem).wait()

        pltpu.sync_copy(idx_hbm, idx_v)

        start_in(0, in0, si0)

        def do_chunk(t, inb, outb):
            bt = t // cpb
            ibase = bt * M

            def g_body(g, carry):
                col = idx_v[pl.ds(ibase + g * _LANES, _LANES)]
                grp = 16
                for r0 in range(0, R, grp):
                    vs = []
                    for r in range(r0, r0 + grp):
                        row = jnp.full((_LANES,), r, jnp.int32)
                        vs.append(plsc.load_gather(inb, [row, col]))
                    for i, r in enumerate(range(r0, r0 + grp)):
                        outb[r, pl.ds(g * _LANES, _LANES)] = vs[i]
                return carry

            lax.fori_loop(0, M // _LANES, g_body, 0)

        def loop_body(t, carry):
            def phase(inb, outb, sin, sout, nxt_in, nxt_sin):
                @pl.when(t >= 2)
                def _():
                    wait_out(t - 2, outb, sout)

                @pl.when(t + 1 < nch)
                def _():
                    start_in(t + 1, nxt_in, nxt_sin)

                wait_in(t, inb, sin)
                do_chunk(t, inb, outb)
                start_out(t, outb, sout)

            @pl.when(t % 2 == 0)
            def _():
                phase(in0, out0, si0, so0, in1, si1)

            @pl.when(t % 2 == 1)
            def _():
                phase(in1, out1, si1, so1, in0, si0)

            return carry

        lax.fori_loop(0, nch, loop_body, 0)
        wait_out(nch - 2, out0, so0)
        wait_out(nch - 1, out1, so1)

    return sc_gather


def kernel(distance):
    original_shape = distance.shape
    d3 = distance.reshape(-1, original_shape[-2], original_shape[-1])
    B, L, M = d3.shape
    idx_flat = _compute_indices(d3)
    gather = _make_sc_gather(B, L, M, R=32)
    out_flat = gather(d3.reshape(B * L, M), idx_flat)
    return out_flat.reshape(original_shape)


# speedup vs baseline: 12.8883x; 1.0021x over previous
import functools

import jax
import jax.numpy as jnp
from jax import lax
from jax.experimental import pallas as pl
from jax.experimental.pallas import tpu as pltpu
from jax.experimental.pallas import tpu_sc as plsc

_NUM_CORES = 2
_NUM_SUBCORES = 16
_LANES = 16
_NW = _NUM_CORES * _NUM_SUBCORES


def _index_body(d_ref, o_ref):
    d = d_ref[0]
    m = d.shape[-1]
    cs = jnp.sum(d, axis=0, keepdims=True)
    w = lax.broadcasted_iota(jnp.int32, (1, m), 1).astype(jnp.float32)
    s1 = jnp.sum(cs * w)
    s0 = jnp.sum(cs)
    scale = (s1 / s0) / jnp.float32(0.5 * m)
    idx = (scale * w).astype(jnp.int32)
    o_ref[0] = jnp.clip(idx, 0, m - 1)


def _compute_indices(distance):
    b, l, m = distance.shape
    out = pl.pallas_call(
        _index_body,
        grid=(b,),
        in_specs=[pl.BlockSpec((1, l, m), lambda i: (i, 0, 0))],
        out_specs=pl.BlockSpec((1, 1, m), lambda i: (i, 0, 0)),
        out_shape=jax.ShapeDtypeStruct((b, 1, m), jnp.int32),
    )(distance)
    return out.reshape(b * m)


def _make_sc_gather(B, L, M, R):
    rows_per_w = L // _NW
    cpb = rows_per_w // R
    nch = B * cpb
    csz = R * M

    mesh = plsc.VectorSubcoreMesh(
        core_axis_name="c", subcore_axis_name="s",
        num_cores=_NUM_CORES, num_subcores=_NUM_SUBCORES)

    @functools.partial(
        pl.kernel,
        mesh=mesh,
        compiler_params=pltpu.CompilerParams(needs_layout_passes=False),
        out_type=jax.ShapeDtypeStruct((B * L, M), jnp.float32),
        scratch_types=[
            pltpu.VMEM((B * M,), jnp.int32),
            pltpu.VMEM((R, M), jnp.float32),
            pltpu.VMEM((R, M), jnp.float32),
            pltpu.VMEM((R, M), jnp.float32),
            pltpu.VMEM((R, M), jnp.float32),
            pltpu.SemaphoreType.DMA,
            pltpu.SemaphoreType.DMA,
            pltpu.SemaphoreType.DMA,
            pltpu.SemaphoreType.DMA,
        ],
    )
    def sc_gather(d_hbm, idx_hbm, out_hbm, idx_v, in0, in1, out0, out1,
                  si0, si1, so0, so1):
        wid = lax.axis_index("s") * _NUM_CORES + lax.axis_index("c")
        row_base = wid * rows_per_w

        def chunk_off(t):
            bt = t // cpb
            ct = t % cpb
            return bt * L + row_base + ct * R

        def start_in(t, buf, sem):
            pltpu.make_async_copy(
                d_hbm.at[pl.ds(chunk_off(t), R)], buf, sem).start()

        def wait_in(t, buf, sem):
            pltpu.make_async_copy(
                d_hbm.at[pl.ds(chunk_off(t), R)], buf, sem).wait()

        def start_out(t, buf, sem):
            pltpu.make_async_copy(
                buf, out_hbm.at[pl.ds(chunk_off(t), R)], sem).start()

        def wait_out(t, buf, sem):
            pltpu.make_async_copy(
                buf, out_hbm.at[pl.ds(chunk_off(t), R)], sem).wait()

        pltpu.sync_copy(idx_hbm, idx_v)

        start_in(0, in0, si0)

        def do_chunk(t, inb, outb):
            bt = t // cpb
            ibase = bt * M

            def g_body(g2, carry):
                grp = 8
                for gg in range(2):
                    g = g2 * 2 + gg
                    col = idx_v[pl.ds(ibase + g * _LANES, _LANES)]
                    for r0 in range(0, R, grp):
                        vs = []
                        for r in range(r0, r0 + grp):
                            row = jnp.full((_LANES,), r, jnp.int32)
                            vs.append(plsc.load_gather(inb, [row, col]))
                        for i, r in enumerate(range(r0, r0 + grp)):
                            outb[r, pl.ds(g * _LANES, _LANES)] = vs[i]
                return carry

            lax.fori_loop(0, M // (2 * _LANES), g_body, 0)

        def loop_body(t, carry):
            def phase(inb, outb, sin, sout, nxt_in, nxt_sin):
                @pl.when(t >= 2)
                def _():
                    wait_out(t - 2, outb, sout)

                @pl.when(t + 1 < nch)
                def _():
                    start_in(t + 1, nxt_in, nxt_sin)

                wait_in(t, inb, sin)
                do_chunk(t, inb, outb)
                start_out(t, outb, sout)

            @pl.when(t % 2 == 0)
            def _():
                phase(in0, out0, si0, so0, in1, si1)

            @pl.when(t % 2 == 1)
            def _():
                phase(in1, out1, si1, so1, in0, si0)

            return carry

        lax.fori_loop(0, nch, loop_body, 0)
        wait_out(nch - 2, out0, so0)
        wait_out(nch - 1, out1, so1)

    return sc_gather


def kernel(distance):
    original_shape = distance.shape
    d3 = distance.reshape(-1, original_shape[-2], original_shape[-1])
    B, L, M = d3.shape
    idx_flat = _compute_indices(d3)
    gather = _make_sc_gather(B, L, M, R=32)
    out_flat = gather(d3.reshape(B * L, M), idx_flat)
    return out_flat.reshape(original_shape)
